# baseline (speedup 1.0000x reference)
import jax
import jax.numpy as jnp
from jax import lax
from jax.experimental import pallas as pl
from jax.experimental.pallas import tpu as pltpu
from jax.experimental.pallas import tpu_sc as plsc

NT = 26
NFF = 4
B = 16384
VD = 100000
VS = 1000000
NC = 2
NS = 16
NW = NC * NS
VC = 2048
NCH_D = 49
NCH_S = 489
DLAST = 48 * VC
SLAST = 488 * VC
DLASTN = 1664
SLASTN = 512
HIST = NCH_S * 16
BKCAP = B + NCH_S * 8 + 16
GW = 877
FCH = 128

_MASK11 = 0x7FF


def _iota16():
    return lax.iota(jnp.int32, 16)


def _full16(v):
    return jnp.full((16,), v, jnp.int32)


def _rsqrt(v):
    y = lax.bitcast_convert_type(v, jnp.int32)
    y = jnp.int32(0x5F3759DF) - (y >> 1)
    r = lax.bitcast_convert_type(y, jnp.float32)
    for _ in range(3):
        r = r * (1.5 - 0.5 * v * r * r)
    return r


def _build_desc():
    import numpy as np
    deep = [(t, ff) for t in range(NT) for ff in range(NFF)]
    desc = np.zeros((NW, 4, 16), np.int32)
    slots = [0] * NW
    for w in range(4):
        desc[w, slots[w], 0:2] = (2, w)
        slots[w] += 1
    for w in range(4, 8):
        desc[w, slots[w], 0:2] = (3, w - 4)
        slots[w] += 1
    di = 0
    for w in range(4, NW):
        while slots[w] < 4 and di < len(deep):
            t, ff = deep[di]
            desc[w, slots[w], 0:3] = (1, t, ff)
            slots[w] += 1
            di += 1
    assert di == len(deep), di
    return desc


def _gather_task(nch, lastn, idx_src, slab_start, slab_wait, last_start,
                 last_wait, tail_gather,
                 row0, U, stage, bkt, hist, fl0, fl1, fs0, fs1, g):
    idx_src()
    zero16 = _full16(0)
    for h in range(HIST // 16):
        hist[pl.ds(h * 16, 16)] = zero16

    def count_body(k, c):
        for r in range(8):
            idx16 = U[r, pl.ds(k * 16, 16)]
            flat16 = (idx16 >> 11) * 16 + _iota16()
            plsc.addupdate_scatter(hist, [flat16], _full16(1))
        return c
    lax.fori_loop(0, 128, count_body, 0)

    def offs_body(b, run):
        row = hist[pl.ds(b * 16, 16)]
        tot = jnp.sum(row)
        ex = plsc.cumsum(row) - row
        hist[pl.ds(b * 16, 16)] = ex + run
        return ((run + tot + 7) >> 3) << 3
    lax.fori_loop(0, nch, offs_body, jnp.int32(0))

    def scat_body(k, c):
        for r in range(8):
            idx16 = U[r, pl.ds(k * 16, 16)]
            pos16 = _full16(r * 2048) + k * 16 + _iota16()
            key16 = (idx16 & _MASK11) | (pos16 << 11)
            flat16 = (idx16 >> 11) * 16 + _iota16()
            dest16 = plsc.load_gather(hist, [flat16])
            plsc.store_scatter(bkt, [dest16], key16)
            plsc.addupdate_scatter(hist, [flat16], _full16(1))
        return c
    lax.fori_loop(0, 128, scat_body, 0)

    nfull = nch - 1
    lasth = nfull & 1

    def resolve(c, colbase, with_tail):
        cm = jnp.maximum(c - 1, 0)
        prev_end = hist[pl.ds(pl.multiple_of(cm * 16, 16), 16)][15]
        start = jnp.where(c == 0, 0, ((prev_end + 7) >> 3) << 3)
        end = hist[pl.ds(pl.multiple_of(c * 16, 16), 16)][15]
        nst = (end - start + 15) >> 4

        def g_body(i, cc):
            gg = start + i * 16
            kp16 = bkt[pl.ds(pl.multiple_of(gg, 8), 16)]
            msk = gg + _iota16() < end
            rel16 = kp16 & _MASK11
            pos16 = lax.shift_right_logical(kp16, 11)
            col16 = rel16 + colbase
            if with_tail:
                tsel = rel16 >= lastn
            for q in range(4):
                vals = []
                for f in (2 * q, 2 * q + 1):
                    v = plsc.bitcast(
                        plsc.load_gather(U, [_full16(f), col16], mask=msk),
                        jnp.float32)
                    if with_tail:
                        tv = tail_gather(f, rel16 - lastn,
                                         jnp.logical_and(msk, tsel))
                        v = jnp.where(tsel, tv, v)
                    vals.append(v)
                w = plsc.bitcast(
                    plsc.pack(vals[0], vals[1],
                              format=plsc.PackFormat.INTERLEAVED), jnp.int32)
                plsc.store_scatter(stage, [_full16(q), pos16], w, mask=msk)
            return cc
        lax.fori_loop(0, nst, g_body, 0)

    slab_start(jnp.int32(0), 0, 0)

    def chunk_body(c, carry):
        h = c & 1
        for par in (0, 1):
            @pl.when(h == par)
            def _(par=par):
                slab_wait(c, par, par)

                @pl.when(c + 1 < nfull)
                def _():
                    slab_start(c + 1, 1 - par, 1 - par)

        resolve(c, h * VC, False)
        return carry
    lax.fori_loop(0, nfull, chunk_body, 0)

    last_start(lasth)
    last_wait(lasth)
    resolve(jnp.int32(nfull), lasth * VC, True)

    def _convert(fl, j):
        for m in range(FCH // 16):
            sl = pl.ds(pl.multiple_of(j * FCH + m * 16, 16), 16)
            for q in range(4):
                y = plsc.bitcast(stage[q, sl], jnp.bfloat16)
                a, b = plsc.unpack(y, format=plsc.PackFormat.INTERLEAVED)
                fl[2 * q, pl.ds(m * 16, 16)] = a
                fl[2 * q + 1, pl.ds(m * 16, 16)] = b

    def _gdst(j):
        return g.at[pl.ds(pl.multiple_of(row0, 8), 8),
                    pl.ds(pl.multiple_of(j * FCH, 128), FCH)]

    def flush_body(j, carry):
        for par, fl, fs in ((0, fl0, fs0), (1, fl1, fs1)):
            @pl.when((j & 1) == par)
            def _():
                @pl.when(j >= 2)
                def _():
                    pltpu.make_async_copy(fl, _gdst(j - 2), fs).wait()
                _convert(fl, j)
                pltpu.make_async_copy(fl, _gdst(j), fs).start()
        return carry
    nfl = B // FCH
    lax.fori_loop(0, nfl, flush_body, 0)
    pltpu.make_async_copy(fl0, _gdst(nfl - 2), fs0).wait()
    pltpu.make_async_copy(fl1, _gdst(nfl - 1), fs1).wait()


def _body(desc, didx4, sidx4, wide, tab4, stab3, dtail, stail, lnw, lnb,
          g, descb, U, stage, bkt, hist, fl0, fl1, tailb, lnv,
          sA, sB, fs0, fs1):
    wid = lax.axis_index("s") * NC + lax.axis_index("c")
    pltpu.sync_copy(desc.at[wid], descb)
    pltpu.sync_copy(lnw, lnv.at[0])
    pltpu.sync_copy(lnb, lnv.at[1])

    def slot_body(slot, carry):
        dv = descb[slot, pl.ds(0, 16)]
        typ = dv[0]
        p0 = dv[1]
        p1 = dv[2]

        @pl.when(typ == 1)
        def _():
            pltpu.sync_copy(dtail.at[p0, p1], tailb)

            def idx_src():
                pltpu.sync_copy(didx4.at[p0], U.at[:, pl.ds(0, 2048)])

            def _dfull(c, h, si):
                return pltpu.make_async_copy(
                    tab4.at[p0, p1, :,
                            pl.ds(pl.multiple_of(c * VC, 128), VC)],
                    U.at[:, pl.ds(h * VC, VC)], sA if si == 0 else sB)

            def _dlast(h):
                return pltpu.make_async_copy(
                    tab4.at[p0, p1, :, pl.ds(DLAST, DLASTN)],
                    U.at[:, pl.ds(h * VC, DLASTN)],
                    sA if h == 0 else sB)

            def slab_start(c, h, si):
                _dfull(c, h, si).start()

            def slab_wait(c, h, si):
                _dfull(c, h, si).wait()

            def last_start(h):
                _dlast(h).start()

            def last_wait(h):
                _dlast(h).wait()

            def tail_gather(f, rel16, msk):
                return plsc.load_gather(tailb, [_full16(f), rel16 & 63],
                                        mask=msk)

            _gather_task(NCH_D, DLASTN, idx_src, slab_start, slab_wait,
                         last_start, last_wait, tail_gather,
                         p0 * 32 + p1 * 8, U, stage, bkt, hist,
                         fl0, fl1, fs0, fs1, g)

        @pl.when(typ == 2)
        def _():
            pltpu.sync_copy(stail.at[p0], tailb)

            def idx_src():
                pltpu.sync_copy(sidx4, U.at[:, pl.ds(0, 2048)])

            def _sfull(c, h, si):
                return pltpu.make_async_copy(
                    stab3.at[p0, :,
                             pl.ds(pl.multiple_of(c * VC, 128), VC)],
                    U.at[:, pl.ds(h * VC, VC)], sA if si == 0 else sB)

            def _slast(h):
                return pltpu.make_async_copy(
                    stab3.at[p0, :, pl.ds(SLAST, SLASTN)],
                    U.at[:, pl.ds(h * VC, SLASTN)],
                    sA if h == 0 else sB)

            def slab_start(c, h, si):
                _sfull(c, h, si).start()

            def slab_wait(c, h, si):
                _sfull(c, h, si).wait()

            def last_start(h):
                _slast(h).start()

            def last_wait(h):
                _slast(h).wait()

            def tail_gather(f, rel16, msk):
                return plsc.load_gather(tailb, [_full16(f), rel16 & 63],
                                        mask=msk)

            _gather_task(NCH_S, SLASTN, idx_src, slab_start, slab_wait,
                         last_start, last_wait, tail_gather,
                         832 + p0 * 8, U, stage, bkt, hist,
                         fl0, fl1, fs0, fs1, g)

        @pl.when(typ == 3)
        def _():
            lnw_v = lnv[0, pl.ds(0, 16)]
            lnb_v = lnv[1, pl.ds(0, 16)]

            def w_body(j, carry2):
                base = pl.multiple_of(p0 * (B // 4) + j * FCH, 128)
                pltpu.sync_copy(wide.at[pl.ds(0, 8), pl.ds(base, FCH)], fl0)
                pltpu.sync_copy(wide.at[pl.ds(8, 5), pl.ds(base, FCH)],
                                fl1.at[pl.ds(0, 5)])

                def ref_f(f):
                    return fl0 if f < 8 else fl1

                def k_body(k, c3):
                    sl = pl.ds(pl.multiple_of(k * 16, 16), 16)
                    xs = [ref_f(f)[f % 8, sl] for f in range(13)]
                    s = xs[0]
                    for f in range(1, 13):
                        s = s + xs[f]
                    mean = s * (1.0 / 13.0)
                    d0 = xs[0] - mean
                    ss = d0 * d0
                    for f in range(1, 13):
                        d = xs[f] - mean
                        ss = ss + d * d
                    r = _rsqrt(ss * (1.0 / 13.0) + 1e-5)
                    for f in range(13):
                        ref_f(f)[f % 8, sl] = ((xs[f] - mean) * r * lnw_v[f]
                                               + lnb_v[f])
                    return c3
                lax.fori_loop(0, FCH // 16, k_body, 0)
                pltpu.sync_copy(fl0, g.at[pl.ds(864, 8), pl.ds(base, FCH)])
                pltpu.sync_copy(fl1.at[pl.ds(0, 5)],
                                g.at[pl.ds(872, 5), pl.ds(base, FCH)])
                return carry2
            lax.fori_loop(0, (B // 4) // FCH, w_body, 0)

        return carry

    lax.fori_loop(0, 4, slot_body, 0)


def kernel(deep_in, wide_in, shared_in, deep_tables, shared_table, ln_w, ln_b):
    desc = jnp.asarray(_build_desc())
    didx4 = deep_in.reshape(NT, 8, 2048)
    sidx4 = shared_in.reshape(8, 2048)
    tab4 = lax.bitcast_convert_type(
        jnp.transpose(deep_tables, (0, 2, 1)).reshape(NT, NFF, 8, VD),
        jnp.int32)
    stab3 = lax.bitcast_convert_type(
        jnp.transpose(shared_table, (1, 0)).reshape(NFF, 8, VS), jnp.int32)
    dtail = jnp.zeros((NT, NFF, 8, 64), jnp.float32).at[:, :, :, :32].set(
        jnp.transpose(deep_tables[:, VD - 32:, :], (0, 2, 1)).reshape(
            NT, NFF, 8, 32))
    stail = jnp.transpose(shared_table[VS - 64:, :], (1, 0)).reshape(
        NFF, 8, 64)
    lnw16 = jnp.zeros((16,), jnp.float32).at[:13].set(ln_w)
    lnb16 = jnp.zeros((16,), jnp.float32).at[:13].set(ln_b)

    mesh = plsc.VectorSubcoreMesh(core_axis_name="c", subcore_axis_name="s")
    run = pl.kernel(
        _body,
        mesh=mesh,
        compiler_params=pltpu.CompilerParams(needs_layout_passes=False),
        out_type=jax.ShapeDtypeStruct((GW, B), jnp.float32),
        scratch_types=[
            pltpu.VMEM((4, 16), jnp.int32),
            pltpu.VMEM((8, 2 * VC), jnp.int32),
            pltpu.VMEM((4, B), jnp.int32),
            pltpu.VMEM((BKCAP,), jnp.int32),
            pltpu.VMEM((HIST,), jnp.int32),
            pltpu.VMEM((8, FCH), jnp.float32),
            pltpu.VMEM((8, FCH), jnp.float32),
            pltpu.VMEM((8, 64), jnp.float32),
            pltpu.VMEM((2, 16), jnp.float32),
            pltpu.SemaphoreType.DMA,
            pltpu.SemaphoreType.DMA,
            pltpu.SemaphoreType.DMA,
            pltpu.SemaphoreType.DMA,
        ],
    )
    g = run(desc, didx4, sidx4, wide_in, tab4, stab3, dtail, stail,
            lnw16, lnb16)
    return jnp.transpose(g, (1, 0))


# speedup vs baseline: 1.0397x; 1.0397x over previous
import jax
import jax.numpy as jnp
from jax import lax
from jax.experimental import pallas as pl
from jax.experimental.pallas import tpu as pltpu
from jax.experimental.pallas import tpu_sc as plsc

NT = 26
NFF = 4
B = 16384
VD = 100000
VS = 1000000
NC = 2
NS = 16
NW = NC * NS
VC = 2048
NCH_D = 49
NCH_S = 489
SHQ = 61
DLAST = 48 * VC
SLAST = 488 * VC
DLASTN = 1664
SLASTN = 512
HIST = NCH_S * 16
BKCAP = B + NCH_S * 8 + 16
GW = 877
FCH = 128
NSLOT = 5

_MASK11 = 0x7FF


def _iota16():
    return lax.iota(jnp.int32, 16)


def _full16(v):
    return jnp.full((16,), v, jnp.int32)


def _rsqrt(v):
    y = lax.bitcast_convert_type(v, jnp.int32)
    y = jnp.int32(0x5F3759DF) - (y >> 1)
    r = lax.bitcast_convert_type(y, jnp.float32)
    for _ in range(3):
        r = r * (1.5 - 0.5 * v * r * r)
    return r


def _build_desc():
    import numpy as np
    desc = np.zeros((NW, NSLOT, 16), np.int32)
    slots = [0] * NW
    for w in range(NW):
        ff, vq = divmod(w, 8)
        desc[w, slots[w], 0:3] = (2, ff, vq)
        slots[w] += 1
    for w in range(4):
        desc[w, slots[w], 0:2] = (3, w)
        slots[w] += 1
    deep = [(t, ff) for t in range(NT) for ff in range(NFF)]
    di = 0
    w = 4
    while di < len(deep):
        if slots[w] < NSLOT:
            t, ff = deep[di]
            desc[w, slots[w], 0:3] = (1, t, ff)
            slots[w] += 1
            di += 1
        w = w + 1 if w + 1 < NW else 0
    return desc


def _gather_task(nfull, lo_b, has_last, zero_stage, lastn, idx_src,
                 slab_start, slab_wait, last_start, last_wait, tail_gather,
                 fstart, fwait, U, stage, bkt, hist, fl0, fl1, fs0, fs1):
    idx_src()
    zero16 = _full16(0)
    for h in range(HIST // 16):
        hist[pl.ds(h * 16, 16)] = zero16
    if zero_stage:
        def z_body(k, c):
            for q in range(4):
                stage[q, pl.ds(pl.multiple_of(k * 16, 16), 16)] = zero16
            return c
        lax.fori_loop(0, B // 16, z_body, 0)

    has_last = jnp.asarray(has_last, bool)
    hi_b = lo_b + nfull + jnp.where(has_last, 1, 0)

    def count_body(k, c):
        for r in range(8):
            idx16 = U[r, pl.ds(k * 16, 16)]
            b16 = idx16 >> 11
            inr = jnp.logical_and(b16 >= lo_b, b16 < hi_b)
            flat16 = b16 * 16 + _iota16()
            plsc.addupdate_scatter(hist, [flat16], _full16(1), mask=inr)
        return c
    lax.fori_loop(0, 128, count_body, 0)

    def offs_body(b, run):
        row = hist[pl.ds(b * 16, 16)]
        tot = jnp.sum(row)
        ex = plsc.cumsum(row) - row
        hist[pl.ds(b * 16, 16)] = ex + run
        return ((run + tot + 7) >> 3) << 3
    lax.fori_loop(lo_b, hi_b, offs_body, jnp.int32(0))

    def scat_body(k, c):
        for r in range(8):
            idx16 = U[r, pl.ds(k * 16, 16)]
            b16 = idx16 >> 11
            inr = jnp.logical_and(b16 >= lo_b, b16 < hi_b)
            pos16 = _full16(r * 2048) + k * 16 + _iota16()
            key16 = (idx16 & _MASK11) | (pos16 << 11)
            flat16 = b16 * 16 + _iota16()
            dest16 = plsc.load_gather(hist, [flat16], mask=inr)
            plsc.store_scatter(bkt, [dest16], key16, mask=inr)
            plsc.addupdate_scatter(hist, [flat16], _full16(1), mask=inr)
        return c
    lax.fori_loop(0, 128, scat_body, 0)

    lasth = nfull & 1

    def resolve(cl, gc, colbase, with_tail):
        gm = jnp.maximum(gc - 1, lo_b)
        prev_end = hist[pl.ds(pl.multiple_of(gm * 16, 16), 16)][15]
        start = jnp.where(cl == 0, 0, ((prev_end + 7) >> 3) << 3)
        end = hist[pl.ds(pl.multiple_of(gc * 16, 16), 16)][15]
        nst = (end - start + 15) >> 4

        def g_body(i, cc):
            gg = start + i * 16
            kp16 = bkt[pl.ds(pl.multiple_of(gg, 8), 16)]
            msk = gg + _iota16() < end
            rel16 = kp16 & _MASK11
            pos16 = lax.shift_right_logical(kp16, 11)
            col16 = rel16 + colbase
            if with_tail:
                tsel = rel16 >= lastn
            for q in range(4):
                vals = []
                for f in (2 * q, 2 * q + 1):
                    v = plsc.bitcast(
                        plsc.load_gather(U, [_full16(f), col16], mask=msk),
                        jnp.float32)
                    if with_tail:
                        tv = tail_gather(f, rel16 - lastn,
                                         jnp.logical_and(msk, tsel))
                        v = jnp.where(tsel, tv, v)
                    vals.append(v)
                w = plsc.bitcast(
                    plsc.pack(vals[0], vals[1],
                              format=plsc.PackFormat.INTERLEAVED), jnp.int32)
                plsc.store_scatter(stage, [_full16(q), pos16], w, mask=msk)
            return cc
        lax.fori_loop(0, nst, g_body, 0)

    slab_start(lo_b + 0, 0, 0)

    def chunk_body(cl, carry):
        h = cl & 1
        gc = lo_b + cl
        for par in (0, 1):
            @pl.when(h == par)
            def _(par=par):
                slab_wait(gc, par, par)

                @pl.when(cl + 1 < nfull)
                def _():
                    slab_start(gc + 1, 1 - par, 1 - par)

        resolve(cl, gc, h * VC, False)
        return carry
    lax.fori_loop(0, nfull, chunk_body, 0)

    @pl.when(has_last)
    def _():
        last_start(lasth)
        last_wait(lasth)
        resolve(jnp.int32(nfull), lo_b + nfull, lasth * VC, True)

    def _convert(fl, j):
        for m in range(FCH // 16):
            sl = pl.ds(pl.multiple_of(j * FCH + m * 16, 16), 16)
            for q in range(4):
                y = plsc.bitcast(stage[q, sl], jnp.bfloat16)
                a, b = plsc.unpack(y, format=plsc.PackFormat.INTERLEAVED)
                fl[2 * q, pl.ds(m * 16, 16)] = a
                fl[2 * q + 1, pl.ds(m * 16, 16)] = b

    def flush_body(j, carry):
        for par, fl, fs in ((0, fl0, fs0), (1, fl1, fs1)):
            @pl.when((j & 1) == par)
            def _(par=par, fl=fl, fs=fs):
                @pl.when(j >= 2)
                def _():
                    fwait(fl, j - 2, fs)
                _convert(fl, j)
                fstart(fl, j, fs)
        return carry
    nfl = B // FCH
    lax.fori_loop(0, nfl, flush_body, 0)
    fwait(fl0, nfl - 2, fs0)
    fwait(fl1, nfl - 1, fs1)


def _body(desc, didx4, sidx4, wide, tab4, stab3, dtail, stail, lnw, lnb,
          g, hh, descb, U, stage, bkt, hist, fl0, fl1, tailb, lnv,
          sA, sB, fs0, fs1):
    wid = lax.axis_index("s") * NC + lax.axis_index("c")
    pltpu.sync_copy(desc.at[wid], descb)
    pltpu.sync_copy(lnw, lnv.at[0])
    pltpu.sync_copy(lnb, lnv.at[1])

    def slot_body(slot, carry):
        dv = descb[slot, pl.ds(0, 16)]
        typ = dv[0]
        p0 = dv[1]
        p1 = dv[2]

        def tail_gather(f, rel16, msk):
            return plsc.load_gather(tailb, [_full16(f), rel16 & 63],
                                    mask=msk)

        @pl.when(typ == 1)
        def _():
            pltpu.sync_copy(dtail.at[p0, p1], tailb)
            row0 = pl.multiple_of(p0 * 32 + p1 * 8, 8)

            def idx_src():
                pltpu.sync_copy(didx4.at[p0], U.at[:, pl.ds(0, 2048)])

            def slab_start(c, h, si):
                pltpu.make_async_copy(
                    tab4.at[p0, p1, :,
                            pl.ds(pl.multiple_of(c * VC, 128), VC)],
                    U.at[:, pl.ds(h * VC, VC)], sA if si == 0 else sB
                ).start()

            def slab_wait(c, h, si):
                pltpu.make_async_copy(
                    tab4.at[p0, p1, :,
                            pl.ds(pl.multiple_of(c * VC, 128), VC)],
                    U.at[:, pl.ds(h * VC, VC)], sA if si == 0 else sB
                ).wait()

            def last_start(h):
                pltpu.make_async_copy(
                    tab4.at[p0, p1, :, pl.ds(DLAST, DLASTN)],
                    U.at[:, pl.ds(h * VC, DLASTN)],
                    sA if h == 0 else sB).start()

            def last_wait(h):
                pltpu.make_async_copy(
                    tab4.at[p0, p1, :, pl.ds(DLAST, DLASTN)],
                    U.at[:, pl.ds(h * VC, DLASTN)],
                    sA if h == 0 else sB).wait()

            def fstart(fl, j, fs):
                pltpu.make_async_copy(
                    fl, g.at[pl.ds(row0, 8),
                             pl.ds(pl.multiple_of(j * FCH, 128), FCH)],
                    fs).start()

            def fwait(fl, j, fs):
                pltpu.make_async_copy(
                    fl, g.at[pl.ds(row0, 8),
                             pl.ds(pl.multiple_of(j * FCH, 128), FCH)],
                    fs).wait()

            _gather_task(48, 0, True, False, DLASTN, idx_src,
                         slab_start, slab_wait, last_start, last_wait,
                         tail_gather, fstart, fwait,
                         U, stage, bkt, hist, fl0, fl1, fs0, fs1)

        @pl.when(typ == 2)
        def _():
            pltpu.sync_copy(stail.at[p0], tailb)
            lo_b = p1 * SHQ
            has_last = p1 == 7
            row0 = pl.multiple_of(832 + p0 * 8, 8)

            def idx_src():
                pltpu.sync_copy(sidx4, U.at[:, pl.ds(0, 2048)])

            def slab_start(c, h, si):
                pltpu.make_async_copy(
                    stab3.at[p0, :,
                             pl.ds(pl.multiple_of(c * VC, 128), VC)],
                    U.at[:, pl.ds(h * VC, VC)], sA if si == 0 else sB
                ).start()

            def slab_wait(c, h, si):
                pltpu.make_async_copy(
                    stab3.at[p0, :,
                             pl.ds(pl.multiple_of(c * VC, 128), VC)],
                    U.at[:, pl.ds(h * VC, VC)], sA if si == 0 else sB
                ).wait()

            def last_start(h):
                pltpu.make_async_copy(
                    stab3.at[p0, :, pl.ds(SLAST, SLASTN)],
                    U.at[:, pl.ds(h * VC, SLASTN)],
                    sA if h == 0 else sB).start()

            def last_wait(h):
                pltpu.make_async_copy(
                    stab3.at[p0, :, pl.ds(SLAST, SLASTN)],
                    U.at[:, pl.ds(h * VC, SLASTN)],
                    sA if h == 0 else sB).wait()

            def fstart(fl, j, fs):
                cols = pl.ds(pl.multiple_of(j * FCH, 128), FCH)

                @pl.when(p1 == 0)
                def _():
                    pltpu.make_async_copy(
                        fl, g.at[pl.ds(row0, 8), cols], fs).start()

                @pl.when(p1 > 0)
                def _():
                    pltpu.make_async_copy(
                        fl, hh.at[p1 - 1, pl.ds(pl.multiple_of(p0 * 8, 8), 8),
                                  cols], fs).start()

            def fwait(fl, j, fs):
                cols = pl.ds(pl.multiple_of(j * FCH, 128), FCH)

                @pl.when(p1 == 0)
                def _():
                    pltpu.make_async_copy(
                        fl, g.at[pl.ds(row0, 8), cols], fs).wait()

                @pl.when(p1 > 0)
                def _():
                    pltpu.make_async_copy(
                        fl, hh.at[p1 - 1, pl.ds(pl.multiple_of(p0 * 8, 8), 8),
                                  cols], fs).wait()

            _gather_task(SHQ, lo_b, has_last, True, SLASTN, idx_src,
                         slab_start, slab_wait, last_start, last_wait,
                         tail_gather, fstart, fwait,
                         U, stage, bkt, hist, fl0, fl1, fs0, fs1)

        @pl.when(typ == 3)
        def _():
            lnw_v = lnv[0, pl.ds(0, 16)]
            lnb_v = lnv[1, pl.ds(0, 16)]

            def w_body(j, carry2):
                base = pl.multiple_of(p0 * (B // 4) + j * FCH, 128)
                pltpu.sync_copy(wide.at[pl.ds(0, 8), pl.ds(base, FCH)], fl0)
                pltpu.sync_copy(wide.at[pl.ds(8, 5), pl.ds(base, FCH)],
                                fl1.at[pl.ds(0, 5)])

                def ref_f(f):
                    return fl0 if f < 8 else fl1

                def k_body(k, c3):
                    sl = pl.ds(pl.multiple_of(k * 16, 16), 16)
                    xs = [ref_f(f)[f % 8, sl] for f in range(13)]
                    s = xs[0]
                    for f in range(1, 13):
                        s = s + xs[f]
                    mean = s * (1.0 / 13.0)
                    d0 = xs[0] - mean
                    ss = d0 * d0
                    for f in range(1, 13):
                        d = xs[f] - mean
                        ss = ss + d * d
                    r = _rsqrt(ss * (1.0 / 13.0) + 1e-5)
                    for f in range(13):
                        ref_f(f)[f % 8, sl] = ((xs[f] - mean) * r * lnw_v[f]
                                               + lnb_v[f])
                    return c3
                lax.fori_loop(0, FCH // 16, k_body, 0)
                pltpu.sync_copy(fl0, g.at[pl.ds(864, 8), pl.ds(base, FCH)])
                pltpu.sync_copy(fl1.at[pl.ds(0, 5)],
                                g.at[pl.ds(872, 5), pl.ds(base, FCH)])
                return carry2
            lax.fori_loop(0, (B // 4) // FCH, w_body, 0)

        return carry

    lax.fori_loop(0, NSLOT, slot_body, 0)


def kernel(deep_in, wide_in, shared_in, deep_tables, shared_table, ln_w, ln_b):
    desc = jnp.asarray(_build_desc())
    didx4 = deep_in.reshape(NT, 8, 2048)
    sidx4 = shared_in.reshape(8, 2048)
    tab4 = lax.bitcast_convert_type(
        jnp.transpose(deep_tables, (0, 2, 1)).reshape(NT, NFF, 8, VD),
        jnp.int32)
    stab3 = lax.bitcast_convert_type(
        jnp.transpose(shared_table, (1, 0)).reshape(NFF, 8, VS), jnp.int32)
    dtail = jnp.zeros((NT, NFF, 8, 64), jnp.float32).at[:, :, :, :32].set(
        jnp.transpose(deep_tables[:, VD - 32:, :], (0, 2, 1)).reshape(
            NT, NFF, 8, 32))
    stail = jnp.transpose(shared_table[VS - 64:, :], (1, 0)).reshape(
        NFF, 8, 64)
    lnw16 = jnp.zeros((16,), jnp.float32).at[:13].set(ln_w)
    lnb16 = jnp.zeros((16,), jnp.float32).at[:13].set(ln_b)

    mesh = plsc.VectorSubcoreMesh(core_axis_name="c", subcore_axis_name="s")
    run = pl.kernel(
        _body,
        mesh=mesh,
        compiler_params=pltpu.CompilerParams(needs_layout_passes=False),
        out_type=(jax.ShapeDtypeStruct((GW, B), jnp.float32),
                  jax.ShapeDtypeStruct((7, 32, B), jnp.float32)),
        scratch_types=[
            pltpu.VMEM((NSLOT, 16), jnp.int32),
            pltpu.VMEM((8, 2 * VC), jnp.int32),
            pltpu.VMEM((4, B), jnp.int32),
            pltpu.VMEM((BKCAP,), jnp.int32),
            pltpu.VMEM((HIST,), jnp.int32),
            pltpu.VMEM((8, FCH), jnp.float32),
            pltpu.VMEM((8, FCH), jnp.float32),
            pltpu.VMEM((8, 64), jnp.float32),
            pltpu.VMEM((2, 16), jnp.float32),
            pltpu.SemaphoreType.DMA,
            pltpu.SemaphoreType.DMA,
            pltpu.SemaphoreType.DMA,
            pltpu.SemaphoreType.DMA,
        ],
    )
    g, hh = run(desc, didx4, sidx4, wide_in, tab4, stab3, dtail, stail,
                lnw16, lnb16)
    out = jnp.transpose(g, (1, 0))
    upd = jnp.transpose(jnp.sum(hh, axis=0), (1, 0))
    return out.at[:, 832:864].add(upd)


# speedup vs baseline: 1.1146x; 1.0720x over previous
import jax
import jax.numpy as jnp
from jax import lax
from jax.experimental import pallas as pl
from jax.experimental.pallas import tpu as pltpu
from jax.experimental.pallas import tpu_sc as plsc

NT = 26
NFF = 4
B = 16384
VD = 100000
VS = 1000000
NC = 2
NS = 16
NW = NC * NS
VC = 2048
NCH_D = 49
NCH_S = 489
SHQ = 61
DLAST = 48 * VC
SLAST = 488 * VC
DLASTN = 1664
SLASTN = 512
HIST = NCH_S * 16
BKCAP = B + NCH_S * 8 + 16
GW = 877
FCH = 128
NSLOT = 5

_MASK11 = 0x7FF


def _iota16():
    return lax.iota(jnp.int32, 16)


def _full16(v):
    return jnp.full((16,), v, jnp.int32)


def _rsqrt(v):
    y = lax.bitcast_convert_type(v, jnp.int32)
    y = jnp.int32(0x5F3759DF) - (y >> 1)
    r = lax.bitcast_convert_type(y, jnp.float32)
    for _ in range(3):
        r = r * (1.5 - 0.5 * v * r * r)
    return r


def _build_desc():
    import numpy as np
    desc = np.zeros((NW, NSLOT, 16), np.int32)
    slots = [0] * NW
    for w in range(NW):
        ff, vq = divmod(w, 8)
        desc[w, slots[w], 0:3] = (2, ff, vq)
        slots[w] += 1
    for w in range(4):
        desc[w, slots[w], 0:2] = (3, w)
        slots[w] += 1
    deep = [(t, ff) for t in range(NT) for ff in range(NFF)]
    di = 0
    w = 4
    while di < len(deep):
        if slots[w] < NSLOT:
            t, ff = deep[di]
            desc[w, slots[w], 0:3] = (1, t, ff)
            slots[w] += 1
            di += 1
        w = w + 1 if w + 1 < NW else 0
    return desc


def _gather_task(nfull, lo_b, has_last, zero_stage, lastn, idx_src,
                 slab_start, slab_wait, last_start, last_wait, tail_gather,
                 fstart, fwait, U, stage, bkt, hist, fl0, fl1, fs0, fs1):
    idx_src()
    zero16 = _full16(0)
    for h in range(HIST // 16):
        hist[pl.ds(h * 16, 16)] = zero16
    if zero_stage:
        def z_body(k, c):
            for q in range(4):
                stage[q, pl.ds(pl.multiple_of(k * 16, 16), 16)] = zero16
            return c
        lax.fori_loop(0, B // 16, z_body, 0)

    has_last = jnp.asarray(has_last, bool)
    hi_b = lo_b + nfull + jnp.where(has_last, 1, 0)

    def count_body(k, c):
        for r in range(8):
            idx16 = U[r, pl.ds(k * 16, 16)]
            b16 = idx16 >> 11
            inr = jnp.logical_and(b16 >= lo_b, b16 < hi_b)
            flat16 = b16 * 16 + _iota16()
            plsc.addupdate_scatter(hist, [flat16], _full16(1), mask=inr)
        return c
    lax.fori_loop(0, 128, count_body, 0)

    def offs_body(b, run):
        row = hist[pl.ds(b * 16, 16)]
        tot = jnp.sum(row)
        ex = plsc.cumsum(row) - row
        hist[pl.ds(b * 16, 16)] = ex + run
        return ((run + tot + 7) >> 3) << 3
    lax.fori_loop(lo_b, hi_b, offs_body, jnp.int32(0))

    def scat_body(k, c):
        for r in range(8):
            idx16 = U[r, pl.ds(k * 16, 16)]
            b16 = idx16 >> 11
            inr = jnp.logical_and(b16 >= lo_b, b16 < hi_b)
            pos16 = _full16(r * 2048) + k * 16 + _iota16()
            key16 = (idx16 & _MASK11) | (pos16 << 11)
            flat16 = b16 * 16 + _iota16()
            dest16 = plsc.load_gather(hist, [flat16], mask=inr)
            plsc.store_scatter(bkt, [dest16], key16, mask=inr)
            plsc.addupdate_scatter(hist, [flat16], _full16(1), mask=inr)
        return c
    lax.fori_loop(0, 128, scat_body, 0)

    lasth = nfull & 1

    def resolve(cl, gc, colbase, with_tail):
        gm = jnp.maximum(gc - 1, lo_b)
        prev_end = hist[pl.ds(pl.multiple_of(gm * 16, 16), 16)][15]
        start = jnp.where(cl == 0, 0, ((prev_end + 7) >> 3) << 3)
        end = hist[pl.ds(pl.multiple_of(gc * 16, 16), 16)][15]
        nst = (end - start + 15) >> 4

        def g_body(i, cc):
            gg = start + i * 16
            kp16 = bkt[pl.ds(pl.multiple_of(gg, 8), 16)]
            msk = gg + _iota16() < end
            rel16 = kp16 & _MASK11
            pos16 = lax.shift_right_logical(kp16, 11)
            col16 = rel16 + colbase
            if with_tail:
                tsel = rel16 >= lastn
            for q in range(4):
                vals = []
                for f in (2 * q, 2 * q + 1):
                    v = plsc.bitcast(
                        plsc.load_gather(U, [_full16(f), col16], mask=msk),
                        jnp.float32)
                    if with_tail:
                        tv = tail_gather(f, rel16 - lastn,
                                         jnp.logical_and(msk, tsel))
                        v = jnp.where(tsel, tv, v)
                    vals.append(v)
                w = plsc.bitcast(
                    plsc.pack(vals[0], vals[1],
                              format=plsc.PackFormat.INTERLEAVED), jnp.int32)
                plsc.store_scatter(stage, [_full16(q), pos16], w, mask=msk)
            return cc
        lax.fori_loop(0, nst, g_body, 0)

    slab_start(lo_b + 0, 0, 0)

    def chunk_body(cl, carry):
        h = cl & 1
        gc = lo_b + cl
        for par in (0, 1):
            @pl.when(h == par)
            def _(par=par):
                @pl.when(cl + 1 < nfull)
                def _():
                    slab_start(gc + 1, 1 - par, 1 - par)

                slab_wait(gc, par, par)

        resolve(cl, gc, h * VC, False)
        return carry
    lax.fori_loop(0, nfull, chunk_body, 0)

    @pl.when(has_last)
    def _():
        last_start(lasth)
        last_wait(lasth)
        resolve(jnp.int32(nfull), lo_b + nfull, lasth * VC, True)

    def _convert(fl, j):
        for m in range(FCH // 16):
            sl = pl.ds(pl.multiple_of(j * FCH + m * 16, 16), 16)
            for q in range(4):
                y = plsc.bitcast(stage[q, sl], jnp.bfloat16)
                a, b = plsc.unpack(y, format=plsc.PackFormat.INTERLEAVED)
                fl[2 * q, pl.ds(m * 16, 16)] = a
                fl[2 * q + 1, pl.ds(m * 16, 16)] = b

    def flush_body(j, carry):
        for par, fl, fs in ((0, fl0, fs0), (1, fl1, fs1)):
            @pl.when((j & 1) == par)
            def _(par=par, fl=fl, fs=fs):
                @pl.when(j >= 2)
                def _():
                    fwait(fl, j - 2, fs)
                _convert(fl, j)
                fstart(fl, j, fs)
        return carry
    nfl = B // FCH
    lax.fori_loop(0, nfl, flush_body, 0)
    fwait(fl0, nfl - 2, fs0)
    fwait(fl1, nfl - 1, fs1)


def _body(desc, didx4, sidx4, wide, tab4, stab3, dtail, stail, lnw, lnb,
          g, hh, descb, U, stage, bkt, hist, fl0, fl1, tailb, lnv,
          sA, sB, fs0, fs1):
    wid = lax.axis_index("s") * NC + lax.axis_index("c")
    pltpu.sync_copy(desc.at[wid], descb)
    pltpu.sync_copy(lnw, lnv.at[0])
    pltpu.sync_copy(lnb, lnv.at[1])

    def slot_body(slot, carry):
        dv = descb[slot, pl.ds(0, 16)]
        typ = dv[0]
        p0 = dv[1]
        p1 = dv[2]

        def tail_gather(f, rel16, msk):
            return plsc.load_gather(tailb, [_full16(f), rel16 & 63],
                                    mask=msk)

        @pl.when(typ == 1)
        def _():
            pltpu.sync_copy(dtail.at[p0, p1], tailb)
            row0 = pl.multiple_of(p0 * 32 + p1 * 8, 8)

            def idx_src():
                pltpu.sync_copy(didx4.at[p0], U.at[:, pl.ds(0, 2048)])

            def slab_start(c, h, si):
                pltpu.make_async_copy(
                    tab4.at[p0, p1, :,
                            pl.ds(pl.multiple_of(c * VC, 128), VC)],
                    U.at[:, pl.ds(h * VC, VC)], sA if si == 0 else sB
                ).start()

            def slab_wait(c, h, si):
                pltpu.make_async_copy(
                    tab4.at[p0, p1, :,
                            pl.ds(pl.multiple_of(c * VC, 128), VC)],
                    U.at[:, pl.ds(h * VC, VC)], sA if si == 0 else sB
                ).wait()

            def last_start(h):
                pltpu.make_async_copy(
                    tab4.at[p0, p1, :, pl.ds(DLAST, DLASTN)],
                    U.at[:, pl.ds(h * VC, DLASTN)],
                    sA if h == 0 else sB).start()

            def last_wait(h):
                pltpu.make_async_copy(
                    tab4.at[p0, p1, :, pl.ds(DLAST, DLASTN)],
                    U.at[:, pl.ds(h * VC, DLASTN)],
                    sA if h == 0 else sB).wait()

            def fstart(fl, j, fs):
                pltpu.make_async_copy(
                    fl, g.at[pl.ds(row0, 8),
                             pl.ds(pl.multiple_of(j * FCH, 128), FCH)],
                    fs).start()

            def fwait(fl, j, fs):
                pltpu.make_async_copy(
                    fl, g.at[pl.ds(row0, 8),
                             pl.ds(pl.multiple_of(j * FCH, 128), FCH)],
                    fs).wait()

            _gather_task(48, 0, True, False, DLASTN, idx_src,
                         slab_start, slab_wait, last_start, last_wait,
                         tail_gather, fstart, fwait,
                         U, stage, bkt, hist, fl0, fl1, fs0, fs1)

        @pl.when(typ == 2)
        def _():
            pltpu.sync_copy(stail.at[p0], tailb)
            lo_b = p1 * SHQ
            has_last = p1 == 7
            row0 = pl.multiple_of(832 + p0 * 8, 8)

            def idx_src():
                pltpu.sync_copy(sidx4, U.at[:, pl.ds(0, 2048)])

            def slab_start(c, h, si):
                pltpu.make_async_copy(
                    stab3.at[p0, :,
                             pl.ds(pl.multiple_of(c * VC, 128), VC)],
                    U.at[:, pl.ds(h * VC, VC)], sA if si == 0 else sB
                ).start()

            def slab_wait(c, h, si):
                pltpu.make_async_copy(
                    stab3.at[p0, :,
                             pl.ds(pl.multiple_of(c * VC, 128), VC)],
                    U.at[:, pl.ds(h * VC, VC)], sA if si == 0 else sB
                ).wait()

            def last_start(h):
                pltpu.make_async_copy(
                    stab3.at[p0, :, pl.ds(SLAST, SLASTN)],
                    U.at[:, pl.ds(h * VC, SLASTN)],
                    sA if h == 0 else sB).start()

            def last_wait(h):
                pltpu.make_async_copy(
                    stab3.at[p0, :, pl.ds(SLAST, SLASTN)],
                    U.at[:, pl.ds(h * VC, SLASTN)],
                    sA if h == 0 else sB).wait()

            def fstart(fl, j, fs):
                cols = pl.ds(pl.multiple_of(j * FCH, 128), FCH)

                @pl.when(p1 == 0)
                def _():
                    pltpu.make_async_copy(
                        fl, g.at[pl.ds(row0, 8), cols], fs).start()

                @pl.when(p1 > 0)
                def _():
                    pltpu.make_async_copy(
                        fl, hh.at[p1 - 1, pl.ds(pl.multiple_of(p0 * 8, 8), 8),
                                  cols], fs).start()

            def fwait(fl, j, fs):
                cols = pl.ds(pl.multiple_of(j * FCH, 128), FCH)

                @pl.when(p1 == 0)
                def _():
                    pltpu.make_async_copy(
                        fl, g.at[pl.ds(row0, 8), cols], fs).wait()

                @pl.when(p1 > 0)
                def _():
                    pltpu.make_async_copy(
                        fl, hh.at[p1 - 1, pl.ds(pl.multiple_of(p0 * 8, 8), 8),
                                  cols], fs).wait()

            _gather_task(SHQ, lo_b, has_last, True, SLASTN, idx_src,
                         slab_start, slab_wait, last_start, last_wait,
                         tail_gather, fstart, fwait,
                         U, stage, bkt, hist, fl0, fl1, fs0, fs1)

        @pl.when(typ == 3)
        def _():
            lnw_v = lnv[0, pl.ds(0, 16)]
            lnb_v = lnv[1, pl.ds(0, 16)]

            def w_body(j, carry2):
                base = pl.multiple_of(p0 * (B // 4) + j * FCH, 128)
                pltpu.sync_copy(wide.at[pl.ds(0, 8), pl.ds(base, FCH)], fl0)
                pltpu.sync_copy(wide.at[pl.ds(8, 5), pl.ds(base, FCH)],
                                fl1.at[pl.ds(0, 5)])

                def ref_f(f):
                    return fl0 if f < 8 else fl1

                def k_body(k, c3):
                    sl = pl.ds(pl.multiple_of(k * 16, 16), 16)
                    xs = [ref_f(f)[f % 8, sl] for f in range(13)]
                    s = xs[0]
                    for f in range(1, 13):
                        s = s + xs[f]
                    mean = s * (1.0 / 13.0)
                    d0 = xs[0] - mean
                    ss = d0 * d0
                    for f in range(1, 13):
                        d = xs[f] - mean
                        ss = ss + d * d
                    r = _rsqrt(ss * (1.0 / 13.0) + 1e-5)
                    for f in range(13):
                        ref_f(f)[f % 8, sl] = ((xs[f] - mean) * r * lnw_v[f]
                                               + lnb_v[f])
                    return c3
                lax.fori_loop(0, FCH // 16, k_body, 0)
                pltpu.sync_copy(fl0, g.at[pl.ds(864, 8), pl.ds(base, FCH)])
                pltpu.sync_copy(fl1.at[pl.ds(0, 5)],
                                g.at[pl.ds(872, 5), pl.ds(base, FCH)])
                return carry2
            lax.fori_loop(0, (B // 4) // FCH, w_body, 0)

        return carry

    lax.fori_loop(0, NSLOT, slot_body, 0)


def kernel(deep_in, wide_in, shared_in, deep_tables, shared_table, ln_w, ln_b):
    desc = jnp.asarray(_build_desc())
    didx4 = deep_in.reshape(NT, 8, 2048)
    sidx4 = shared_in.reshape(8, 2048)
    tab4 = lax.bitcast_convert_type(
        jnp.transpose(deep_tables, (0, 2, 1)).reshape(NT, NFF, 8, VD),
        jnp.int32)
    stab3 = lax.bitcast_convert_type(
        jnp.transpose(shared_table, (1, 0)).reshape(NFF, 8, VS), jnp.int32)
    dtail = jnp.zeros((NT, NFF, 8, 64), jnp.float32).at[:, :, :, :32].set(
        jnp.transpose(deep_tables[:, VD - 32:, :], (0, 2, 1)).reshape(
            NT, NFF, 8, 32))
    stail = jnp.transpose(shared_table[VS - 64:, :], (1, 0)).reshape(
        NFF, 8, 64)
    lnw16 = jnp.zeros((16,), jnp.float32).at[:13].set(ln_w)
    lnb16 = jnp.zeros((16,), jnp.float32).at[:13].set(ln_b)

    mesh = plsc.VectorSubcoreMesh(core_axis_name="c", subcore_axis_name="s")
    run = pl.kernel(
        _body,
        mesh=mesh,
        compiler_params=pltpu.CompilerParams(needs_layout_passes=False),
        out_type=(jax.ShapeDtypeStruct((GW, B), jnp.float32),
                  jax.ShapeDtypeStruct((7, 32, B), jnp.float32)),
        scratch_types=[
            pltpu.VMEM((NSLOT, 16), jnp.int32),
            pltpu.VMEM((8, 2 * VC), jnp.int32),
            pltpu.VMEM((4, B), jnp.int32),
            pltpu.VMEM((BKCAP,), jnp.int32),
            pltpu.VMEM((HIST,), jnp.int32),
            pltpu.VMEM((8, FCH), jnp.float32),
            pltpu.VMEM((8, FCH), jnp.float32),
            pltpu.VMEM((8, 64), jnp.float32),
            pltpu.VMEM((2, 16), jnp.float32),
            pltpu.SemaphoreType.DMA,
            pltpu.SemaphoreType.DMA,
            pltpu.SemaphoreType.DMA,
            pltpu.SemaphoreType.DMA,
        ],
    )
    g, hh = run(desc, didx4, sidx4, wide_in, tab4, stab3, dtail, stail,
                lnw16, lnb16)
    out = jnp.transpose(g, (1, 0))
    upd = jnp.transpose(jnp.sum(hh, axis=0), (1, 0))
    return out.at[:, 832:864].add(upd)
